# Initial kernel scaffold; baseline (speedup 1.0000x reference)
import functools

import jax
import jax.numpy as jnp
from jax import lax
from jax.experimental import pallas as pl
from jax.experimental.pallas import tpu as pltpu
from jax.experimental.pallas import tpu_sc as plsc

_N = 10000
_E = 320000
_H = 128
_L = 15

_NC = 2
_NS = 16
_NW = _NC * _NS
_EPW = _E // _NW
_CH = 80
_NCH = _EPW // _CH
_RPT = _N // _NS

_mesh = plsc.VectorSubcoreMesh(core_axis_name="c", subcore_axis_name="s")


def _wid():
    return lax.axis_index("s") * _NC + lax.axis_index("c")


@functools.partial(
    pl.kernel,
    out_type=(jax.ShapeDtypeStruct((_E, _H), jnp.float32),
              jax.ShapeDtypeStruct((_E, _H), jnp.float32)),
    mesh=_mesh,
    scratch_types=[
        pltpu.VMEM((_NCH, _CH), jnp.int32),
        pltpu.VMEM((_NCH, _CH), jnp.int32),
        pltpu.VMEM((_CH, _H), jnp.float32),
        pltpu.VMEM((_CH, _H), jnp.float32),
        pltpu.SemaphoreType.DMA,
        pltpu.SemaphoreType.DMA,
    ],
)
def _sc_gather(xs_hbm, xd_hbm, src_hbm, dst_hbm, gs_hbm, gd_hbm,
               sidx, didx, rows_s, rows_d, sem_s, sem_d):
    w = _wid()
    pltpu.sync_copy(src_hbm.at[pl.ds(w * _NCH, _NCH)], sidx)
    pltpu.sync_copy(dst_hbm.at[pl.ds(w * _NCH, _NCH)], didx)
    base = w * _EPW

    def body(j, carry):
        cs = pltpu.async_copy(xs_hbm.at[sidx.at[j]], rows_s, sem_s)
        cd = pltpu.async_copy(xd_hbm.at[didx.at[j]], rows_d, sem_d)
        cs.wait()
        pltpu.sync_copy(rows_s, gs_hbm.at[pl.ds(base + j * _CH, _CH)])
        cd.wait()
        pltpu.sync_copy(rows_d, gd_hbm.at[pl.ds(base + j * _CH, _CH)])
        return carry

    lax.fori_loop(0, _NCH, body, 0)


@functools.partial(
    pl.kernel,
    out_type=jax.ShapeDtypeStruct((_NC, _N, _H), jnp.float32),
    mesh=_mesh,
    scratch_types=[
        pltpu.VMEM((_NCH, _CH), jnp.int32),
        pltpu.VMEM((_CH, _H), jnp.float32),
        pltpu.VMEM_SHARED((_N, _H), jnp.float32),
        pltpu.SemaphoreType.DMA,
    ],
)
def _sc_scatter(e_hbm, dst_hbm, zeros_hbm, out_hbm, didx, rows, acc, sem):
    c = lax.axis_index("c")
    s = lax.axis_index("s")
    w = s * _NC + c
    pltpu.sync_copy(zeros_hbm.at[pl.ds(s * _RPT, _RPT)],
                    acc.at[pl.ds(s * _RPT, _RPT)])
    pltpu.sync_copy(dst_hbm.at[pl.ds(w * _NCH, _NCH)], didx)
    plsc.subcore_barrier()
    base = w * _EPW

    def body(j, carry):
        pltpu.sync_copy(e_hbm.at[pl.ds(base + j * _CH, _CH)], rows)
        pltpu.sync_copy(rows, acc.at[didx.at[j]], add=True)
        return carry

    lax.fori_loop(0, _NCH, body, 0)
    plsc.subcore_barrier()
    pltpu.sync_copy(acc.at[pl.ds(s * _RPT, _RPT)],
                    out_hbm.at[c, pl.ds(s * _RPT, _RPT)])


def _ln(y, g, b):
    mu = jnp.mean(y, axis=-1, keepdims=True)
    v = jnp.mean((y - mu) ** 2, axis=-1, keepdims=True)
    return (y - mu) * lax.rsqrt(v + 1e-5) * g + b


def _dot(a, b):
    return jnp.dot(a, b, preferred_element_type=jnp.float32)


_BE = 2560
_BN = 2000


def _row_spec(blk, width):
    return pl.BlockSpec((blk, width), lambda i: (i, 0))


def _rep_spec(shape):
    nd = len(shape)
    return pl.BlockSpec(shape, lambda i: (0,) * nd)


def _enc_e_body(ea, W1, b1, W2, b2, g, bt, out):
    h = jnp.maximum(_dot(ea[...], W1[...]) + b1[...], 0.0)
    y = _dot(h, W2[...]) + b2[...]
    out[...] = _ln(y, g[...], bt[...])


def _enc_n_body(na, W1, b1, W2, b2, g, bt, W1s, W1d, x_out, xs_out, xd_out):
    h = jnp.maximum(_dot(na[...], W1[...]) + b1[...], 0.0)
    y = _dot(h, W2[...]) + b2[...]
    x = _ln(y, g[...], bt[...])
    x_out[...] = x
    xs_out[...] = _dot(x, W1s[...])
    xd_out[...] = _dot(x, W1d[...])


def _edge_mlp_body(e, gs, gd, W1e, b1, W2, b2, g, bt, out):
    ev = e[...]
    h = jnp.maximum(_dot(ev, W1e[...]) + gs[...] + gd[...] + b1[...], 0.0)
    y = _dot(h, W2[...]) + b2[...]
    out[...] = ev + _ln(y, g[...], bt[...])


def _node_mlp_next_body(x, a0, a1, W1x, W1a, b1, W2, b2, g, bt, W1s, W1d,
                        x_out, xs_out, xd_out):
    xv = x[...]
    agg = a0[...] + a1[...]
    h = jnp.maximum(_dot(xv, W1x[...]) + _dot(agg, W1a[...]) + b1[...], 0.0)
    y = _dot(h, W2[...]) + b2[...]
    xn = xv + _ln(y, g[...], bt[...])
    x_out[...] = xn
    xs_out[...] = _dot(xn, W1s[...])
    xd_out[...] = _dot(xn, W1d[...])


def _node_mlp_last_body(x, a0, a1, W1x, W1a, b1, W2, b2, g, bt, x_out):
    xv = x[...]
    agg = a0[...] + a1[...]
    h = jnp.maximum(_dot(xv, W1x[...]) + _dot(agg, W1a[...]) + b1[...], 0.0)
    y = _dot(h, W2[...]) + b2[...]
    x_out[...] = xv + _ln(y, g[...], bt[...])


def _dec_body(x, W1, b1, W2, b2, out):
    h = jnp.maximum(_dot(x[...], W1[...]) + b1[...], 0.0)
    out[...] = _dot(h, W2[...]) + b2[...]


def _f32(shape):
    return jax.ShapeDtypeStruct(shape, jnp.float32)


def _row1(v):
    return v.reshape(1, -1)


def kernel(node_attr, edge_attr, edge_index, params):
    p = params
    src2 = edge_index[0].reshape(_NW * _NCH, _CH)
    dst2 = edge_index[1].reshape(_NW * _NCH, _CH)
    zeros = jnp.zeros((_N, _H), jnp.float32)

    wspec = _rep_spec((_H, _H))
    vspec = _rep_spec((1, _H))

    enc_n = pl.pallas_call(
        _enc_n_body,
        grid=(_N // _BN,),
        in_specs=[_row_spec(_BN, _H)] + [wspec, vspec, wspec, vspec, vspec,
                                         vspec, wspec, wspec],
        out_specs=[_row_spec(_BN, _H)] * 3,
        out_shape=[_f32((_N, _H))] * 3,
    )
    x, xs, xd = enc_n(
        node_attr,
        p['enc_n_W1'], _row1(p['enc_n_b1']), p['enc_n_W2'], _row1(p['enc_n_b2']),
        _row1(p['enc_n_g']), _row1(p['enc_n_beta']),
        p['pe_W1'][0, _H:2 * _H], p['pe_W1'][0, 2 * _H:],
    )

    enc_e = pl.pallas_call(
        _enc_e_body,
        grid=(_E // _BE,),
        in_specs=[_row_spec(_BE, 4), _rep_spec((4, _H)), vspec, wspec, vspec,
                  vspec, vspec],
        out_specs=_row_spec(_BE, _H),
        out_shape=_f32((_E, _H)),
    )
    e = enc_e(
        edge_attr,
        p['enc_e_W1'], _row1(p['enc_e_b1']), p['enc_e_W2'], _row1(p['enc_e_b2']),
        _row1(p['enc_e_g']), _row1(p['enc_e_beta']),
    )

    edge_mlp = pl.pallas_call(
        _edge_mlp_body,
        grid=(_E // _BE,),
        in_specs=[_row_spec(_BE, _H)] * 3 + [wspec, vspec, wspec, vspec,
                                             vspec, vspec],
        out_specs=_row_spec(_BE, _H),
        out_shape=_f32((_E, _H)),
    )
    node_mlp_next = pl.pallas_call(
        _node_mlp_next_body,
        grid=(_N // _BN,),
        in_specs=[_row_spec(_BN, _H)] * 3 + [wspec, wspec, vspec, wspec,
                                             vspec, vspec, vspec, wspec, wspec],
        out_specs=[_row_spec(_BN, _H)] * 3,
        out_shape=[_f32((_N, _H))] * 3,
    )
    node_mlp_last = pl.pallas_call(
        _node_mlp_last_body,
        grid=(_N // _BN,),
        in_specs=[_row_spec(_BN, _H)] * 3 + [wspec, wspec, vspec, wspec,
                                             vspec, vspec, vspec],
        out_specs=_row_spec(_BN, _H),
        out_shape=_f32((_N, _H)),
    )

    for i in range(_L):
        gs, gd = _sc_gather(xs, xd, src2, dst2)
        e = edge_mlp(
            e, gs, gd,
            p['pe_W1'][i, :_H], _row1(p['pe_b1'][i]),
            p['pe_W2'][i], _row1(p['pe_b2'][i]),
            _row1(p['pe_g'][i]), _row1(p['pe_beta'][i]),
        )
        agg2 = _sc_scatter(e, dst2, zeros)
        if i < _L - 1:
            x, xs, xd = node_mlp_next(
                x, agg2[0], agg2[1],
                p['pn_W1'][i, :_H], p['pn_W1'][i, _H:], _row1(p['pn_b1'][i]),
                p['pn_W2'][i], _row1(p['pn_b2'][i]),
                _row1(p['pn_g'][i]), _row1(p['pn_beta'][i]),
                p['pe_W1'][i + 1, _H:2 * _H], p['pe_W1'][i + 1, 2 * _H:],
            )
        else:
            x = node_mlp_last(
                x, agg2[0], agg2[1],
                p['pn_W1'][i, :_H], p['pn_W1'][i, _H:], _row1(p['pn_b1'][i]),
                p['pn_W2'][i], _row1(p['pn_b2'][i]),
                _row1(p['pn_g'][i]), _row1(p['pn_beta'][i]),
            )

    dec = pl.pallas_call(
        _dec_body,
        grid=(_N // _BN,),
        in_specs=[_row_spec(_BN, _H), wspec, vspec, _rep_spec((_H, 3)),
                  _rep_spec((1, 3))],
        out_specs=_row_spec(_BN, 3),
        out_shape=_f32((_N, 3)),
    )
    return dec(x, p['dec_W1'], _row1(p['dec_b1']), p['dec_W2'],
               _row1(p['dec_b2']))


# speedup vs baseline: 3.4035x; 3.4035x over previous
import functools

import jax
import jax.numpy as jnp
from jax import lax
from jax.experimental import pallas as pl
from jax.experimental.pallas import tpu as pltpu
from jax.experimental.pallas import tpu_sc as plsc

_N = 10000
_E = 320000
_H = 128
_L = 15

_NC = 2
_NS = 16
_NW = _NC * _NS
_EPW = _E // _NW
_CH = 80
_NCH = _EPW // _CH
_NP = 10240
_RPT = _NP // _NS

_mesh = plsc.VectorSubcoreMesh(core_axis_name="c", subcore_axis_name="s")


def _wid():
    return lax.axis_index("s") * _NC + lax.axis_index("c")


@functools.partial(
    pl.kernel,
    out_type=(jax.ShapeDtypeStruct((_E, _H), jnp.float32),
              jax.ShapeDtypeStruct((_E, _H), jnp.float32)),
    mesh=_mesh,
    scratch_types=[
        pltpu.VMEM((_NCH, _CH), jnp.int32),
        pltpu.VMEM((_NCH, _CH), jnp.int32),
        pltpu.VMEM((_CH, _H), jnp.float32),
        pltpu.VMEM((_CH, _H), jnp.float32),
        pltpu.SemaphoreType.DMA,
        pltpu.SemaphoreType.DMA,
    ],
)
def _sc_gather(xs_hbm, xd_hbm, src_hbm, dst_hbm, gs_hbm, gd_hbm,
               sidx, didx, rows_s, rows_d, sem_s, sem_d):
    w = _wid()
    pltpu.sync_copy(src_hbm.at[w], sidx)
    pltpu.sync_copy(dst_hbm.at[w], didx)
    base = w * _EPW

    def body(j, carry):
        cs = pltpu.async_copy(xs_hbm.at[sidx.at[j]], rows_s, sem_s)
        cd = pltpu.async_copy(xd_hbm.at[didx.at[j]], rows_d, sem_d)
        cs.wait()
        pltpu.sync_copy(rows_s, gs_hbm.at[pl.ds(base + j * _CH, _CH)])
        cd.wait()
        pltpu.sync_copy(rows_d, gd_hbm.at[pl.ds(base + j * _CH, _CH)])
        return carry

    lax.fori_loop(0, _NCH, body, 0)


@functools.partial(
    pl.kernel,
    out_type=jax.ShapeDtypeStruct((_NC, _NP, _H), jnp.float32),
    mesh=_mesh,
    scratch_types=[
        pltpu.VMEM((_NCH, _CH), jnp.int32),
        pltpu.VMEM((_CH, _H), jnp.float32),
        pltpu.VMEM_SHARED((_NP, _H), jnp.float32),
        pltpu.SemaphoreType.DMA,
    ],
)
def _sc_scatter(e_hbm, dst_hbm, zeros_hbm, out_hbm, didx, rows, acc, sem):
    c = lax.axis_index("c")
    s = lax.axis_index("s")
    w = s * _NC + c
    pltpu.sync_copy(zeros_hbm.at[pl.ds(s * _RPT, _RPT)],
                    acc.at[pl.ds(s * _RPT, _RPT)])
    pltpu.sync_copy(dst_hbm.at[w], didx)
    plsc.subcore_barrier()
    base = w * _EPW

    def body(j, carry):
        pltpu.sync_copy(e_hbm.at[pl.ds(base + j * _CH, _CH)], rows)
        pltpu.sync_copy(rows, acc.at[didx.at[j]], add=True)
        return carry

    lax.fori_loop(0, _NCH, body, 0)
    plsc.subcore_barrier()
    pltpu.sync_copy(acc.at[pl.ds(s * _RPT, _RPT)],
                    out_hbm.at[c, pl.ds(s * _RPT, _RPT)])


def _ln(y, g, b):
    mu = jnp.mean(y, axis=-1, keepdims=True)
    v = jnp.mean((y - mu) ** 2, axis=-1, keepdims=True)
    return (y - mu) * lax.rsqrt(v + 1e-5) * g + b


def _dot(a, b):
    return jnp.dot(a, b, preferred_element_type=jnp.float32)


_BE = 2560
_BN = 2000


def _row_spec(blk, width):
    return pl.BlockSpec((blk, width), lambda i: (i, 0))


def _rep_spec(shape):
    nd = len(shape)
    return pl.BlockSpec(shape, lambda i: (0,) * nd)


def _enc_e_body(ea, W1, b1, W2, b2, g, bt, out):
    h = jnp.maximum(_dot(ea[...], W1[...]) + b1[...], 0.0)
    y = _dot(h, W2[...]) + b2[...]
    out[...] = _ln(y, g[...], bt[...])


def _enc_n_body(na, W1, b1, W2, b2, g, bt, W1s, W1d, x_out, xs_out, xd_out):
    h = jnp.maximum(_dot(na[...], W1[...]) + b1[...], 0.0)
    y = _dot(h, W2[...]) + b2[...]
    x = _ln(y, g[...], bt[...])
    x_out[...] = x
    xs_out[...] = _dot(x, W1s[...])
    xd_out[...] = _dot(x, W1d[...])


def _edge_mlp_body(e, gs, gd, W1e, b1, W2, b2, g, bt, out):
    ev = e[...]
    h = jnp.maximum(_dot(ev, W1e[...]) + gs[...] + gd[...] + b1[...], 0.0)
    y = _dot(h, W2[...]) + b2[...]
    out[...] = ev + _ln(y, g[...], bt[...])


def _node_mlp_next_body(x, a0, a1, W1x, W1a, b1, W2, b2, g, bt, W1s, W1d,
                        x_out, xs_out, xd_out):
    xv = x[...]
    agg = a0[...] + a1[...]
    h = jnp.maximum(_dot(xv, W1x[...]) + _dot(agg, W1a[...]) + b1[...], 0.0)
    y = _dot(h, W2[...]) + b2[...]
    xn = xv + _ln(y, g[...], bt[...])
    x_out[...] = xn
    xs_out[...] = _dot(xn, W1s[...])
    xd_out[...] = _dot(xn, W1d[...])


def _node_mlp_last_body(x, a0, a1, W1x, W1a, b1, W2, b2, g, bt, x_out):
    xv = x[...]
    agg = a0[...] + a1[...]
    h = jnp.maximum(_dot(xv, W1x[...]) + _dot(agg, W1a[...]) + b1[...], 0.0)
    y = _dot(h, W2[...]) + b2[...]
    x_out[...] = xv + _ln(y, g[...], bt[...])


def _dec_body(x, W1, b1, W2, b2, out):
    h = jnp.maximum(_dot(x[...], W1[...]) + b1[...], 0.0)
    out[...] = _dot(h, W2[...]) + b2[...]


def _f32(shape):
    return jax.ShapeDtypeStruct(shape, jnp.float32)


def _row1(v):
    return v.reshape(1, -1)


def kernel(node_attr, edge_attr, edge_index, params):
    p = params
    src2 = edge_index[0].reshape(_NW, _NCH, _CH)
    dst2 = edge_index[1].reshape(_NW, _NCH, _CH)
    zeros = jnp.zeros((_NP, _H), jnp.float32)

    wspec = _rep_spec((_H, _H))
    vspec = _rep_spec((1, _H))

    enc_n = pl.pallas_call(
        _enc_n_body,
        grid=(_N // _BN,),
        in_specs=[_row_spec(_BN, _H)] + [wspec, vspec, wspec, vspec, vspec,
                                         vspec, wspec, wspec],
        out_specs=[_row_spec(_BN, _H)] * 3,
        out_shape=[_f32((_N, _H))] * 3,
    )
    x, xs, xd = enc_n(
        node_attr,
        p['enc_n_W1'], _row1(p['enc_n_b1']), p['enc_n_W2'], _row1(p['enc_n_b2']),
        _row1(p['enc_n_g']), _row1(p['enc_n_beta']),
        p['pe_W1'][0, _H:2 * _H], p['pe_W1'][0, 2 * _H:],
    )

    enc_e = pl.pallas_call(
        _enc_e_body,
        grid=(_E // _BE,),
        in_specs=[_row_spec(_BE, 4), _rep_spec((4, _H)), vspec, wspec, vspec,
                  vspec, vspec],
        out_specs=_row_spec(_BE, _H),
        out_shape=_f32((_E, _H)),
    )
    e = enc_e(
        edge_attr,
        p['enc_e_W1'], _row1(p['enc_e_b1']), p['enc_e_W2'], _row1(p['enc_e_b2']),
        _row1(p['enc_e_g']), _row1(p['enc_e_beta']),
    )

    edge_mlp = pl.pallas_call(
        _edge_mlp_body,
        grid=(_E // _BE,),
        in_specs=[_row_spec(_BE, _H)] * 3 + [wspec, vspec, wspec, vspec,
                                             vspec, vspec],
        out_specs=_row_spec(_BE, _H),
        out_shape=_f32((_E, _H)),
    )
    node_mlp_next = pl.pallas_call(
        _node_mlp_next_body,
        grid=(_N // _BN,),
        in_specs=[_row_spec(_BN, _H)] * 3 + [wspec, wspec, vspec, wspec,
                                             vspec, vspec, vspec, wspec, wspec],
        out_specs=[_row_spec(_BN, _H)] * 3,
        out_shape=[_f32((_N, _H))] * 3,
    )
    node_mlp_last = pl.pallas_call(
        _node_mlp_last_body,
        grid=(_N // _BN,),
        in_specs=[_row_spec(_BN, _H)] * 3 + [wspec, wspec, vspec, wspec,
                                             vspec, vspec, vspec],
        out_specs=_row_spec(_BN, _H),
        out_shape=_f32((_N, _H)),
    )

    for i in range(_L):
        gs, gd = _sc_gather(xs, xd, src2, dst2)
        e = edge_mlp(
            e, gs, gd,
            p['pe_W1'][i, :_H], _row1(p['pe_b1'][i]),
            p['pe_W2'][i], _row1(p['pe_b2'][i]),
            _row1(p['pe_g'][i]), _row1(p['pe_beta'][i]),
        )
        agg2 = _sc_scatter(e, dst2, zeros)
        if i < _L - 1:
            x, xs, xd = node_mlp_next(
                x, agg2[0, :_N], agg2[1, :_N],
                p['pn_W1'][i, :_H], p['pn_W1'][i, _H:], _row1(p['pn_b1'][i]),
                p['pn_W2'][i], _row1(p['pn_b2'][i]),
                _row1(p['pn_g'][i]), _row1(p['pn_beta'][i]),
                p['pe_W1'][i + 1, _H:2 * _H], p['pe_W1'][i + 1, 2 * _H:],
            )
        else:
            x = node_mlp_last(
                x, agg2[0, :_N], agg2[1, :_N],
                p['pn_W1'][i, :_H], p['pn_W1'][i, _H:], _row1(p['pn_b1'][i]),
                p['pn_W2'][i], _row1(p['pn_b2'][i]),
                _row1(p['pn_g'][i]), _row1(p['pn_beta'][i]),
            )

    dec = pl.pallas_call(
        _dec_body,
        grid=(_N // _BN,),
        in_specs=[_row_spec(_BN, _H), wspec, vspec, _rep_spec((_H, 3)),
                  _rep_spec((1, 3))],
        out_specs=_row_spec(_BN, 3),
        out_shape=_f32((_N, 3)),
    )
    return dec(x, p['dec_W1'], _row1(p['dec_b1']), p['dec_W2'],
               _row1(p['dec_b2']))


# speedup vs baseline: 4.0059x; 1.1770x over previous
import functools

import jax
import jax.numpy as jnp
from jax import lax
from jax.experimental import pallas as pl
from jax.experimental.pallas import tpu as pltpu
from jax.experimental.pallas import tpu_sc as plsc

_N = 10000
_E = 320000
_H = 128
_L = 15

_NC = 2
_NS = 16
_NW = _NC * _NS
_EPW = _E // _NW
_CH = 80
_NCH = _EPW // _CH
_NP = 10240
_RPT = _NP // _NS

_mesh = plsc.VectorSubcoreMesh(core_axis_name="c", subcore_axis_name="s")


def _wid():
    return lax.axis_index("s") * _NC + lax.axis_index("c")


@functools.partial(
    pl.kernel,
    out_type=(jax.ShapeDtypeStruct((_E, _H), jnp.float32),
              jax.ShapeDtypeStruct((_E, _H), jnp.float32)),
    mesh=_mesh,
    scratch_types=[
        pltpu.VMEM((_NCH, _CH), jnp.int32),
        pltpu.VMEM((_NCH, _CH), jnp.int32),
        pltpu.VMEM((_CH, _H), jnp.float32),
        pltpu.VMEM((_CH, _H), jnp.float32),
        pltpu.VMEM((_CH, _H), jnp.float32),
        pltpu.VMEM((_CH, _H), jnp.float32),
        pltpu.SemaphoreType.DMA,
        pltpu.SemaphoreType.DMA,
    ],
)
def _sc_gather(xs_hbm, xd_hbm, src_hbm, dst_hbm, gs_hbm, gd_hbm,
               sidx, didx, rs0, rd0, rs1, rd1, gsem, wsem):
    w = _wid()
    pltpu.sync_copy(src_hbm.at[w], sidx)
    pltpu.sync_copy(dst_hbm.at[w], didx)
    base = w * _EPW

    def chunk_gather(j, rs, rd):
        cs = pltpu.async_copy(xs_hbm.at[sidx.at[j]], rs, gsem)
        cd = pltpu.async_copy(xd_hbm.at[didx.at[j]], rd, gsem)
        return cs, cd

    def chunk_write(j, rs, rd):
        ws = pltpu.async_copy(rs, gs_hbm.at[pl.ds(base + j * _CH, _CH)], wsem)
        wd = pltpu.async_copy(rd, gd_hbm.at[pl.ds(base + j * _CH, _CH)], wsem)
        return ws, wd

    def body(i, carry):
        j0 = i * 2
        j1 = j0 + 1
        cs0, cd0 = chunk_gather(j0, rs0, rd0)
        cs1, cd1 = chunk_gather(j1, rs1, rd1)
        cs0.wait()
        cd0.wait()
        ws0, wd0 = chunk_write(j0, rs0, rd0)
        cs1.wait()
        cd1.wait()
        ws1, wd1 = chunk_write(j1, rs1, rd1)
        ws0.wait()
        wd0.wait()
        ws1.wait()
        wd1.wait()
        return carry

    lax.fori_loop(0, _NCH // 2, body, 0)
    jl = _NCH - 1
    cs, cd = chunk_gather(jl, rs0, rd0)
    cs.wait()
    cd.wait()
    ws, wd = chunk_write(jl, rs0, rd0)
    ws.wait()
    wd.wait()


@functools.partial(
    pl.kernel,
    out_type=jax.ShapeDtypeStruct((_NC, _NP, _H), jnp.float32),
    mesh=_mesh,
    scratch_types=[
        pltpu.VMEM((_NCH, _CH), jnp.int32),
        pltpu.VMEM((_CH, _H), jnp.float32),
        pltpu.VMEM((_CH, _H), jnp.float32),
        pltpu.VMEM_SHARED((_NP, _H), jnp.float32),
        pltpu.SemaphoreType.DMA,
        pltpu.SemaphoreType.DMA,
    ],
)
def _sc_scatter(e_hbm, dst_hbm, zeros_hbm, out_hbm, didx, rows0, rows1, acc,
                rsem, asem):
    c = lax.axis_index("c")
    s = lax.axis_index("s")
    w = s * _NC + c
    pltpu.sync_copy(zeros_hbm.at[pl.ds(s * _RPT, _RPT)],
                    acc.at[pl.ds(s * _RPT, _RPT)])
    pltpu.sync_copy(dst_hbm.at[w], didx)
    plsc.subcore_barrier()
    base = w * _EPW

    def body(i, carry):
        j0 = i * 2
        j1 = j0 + 1
        r0 = pltpu.async_copy(e_hbm.at[pl.ds(base + j0 * _CH, _CH)], rows0, rsem)
        r1 = pltpu.async_copy(e_hbm.at[pl.ds(base + j1 * _CH, _CH)], rows1, rsem)
        r0.wait()
        a0 = pltpu.async_copy(rows0, acc.at[didx.at[j0]], asem, add=True)
        r1.wait()
        a1 = pltpu.async_copy(rows1, acc.at[didx.at[j1]], asem, add=True)
        a0.wait()
        a1.wait()
        return carry

    lax.fori_loop(0, _NCH // 2, body, 0)
    jl = _NCH - 1
    rl = pltpu.async_copy(e_hbm.at[pl.ds(base + jl * _CH, _CH)], rows0, rsem)
    rl.wait()
    al = pltpu.async_copy(rows0, acc.at[didx.at[jl]], asem, add=True)
    al.wait()
    plsc.subcore_barrier()
    pltpu.sync_copy(acc.at[pl.ds(s * _RPT, _RPT)],
                    out_hbm.at[c, pl.ds(s * _RPT, _RPT)])


def _ln(y, g, b):
    mu = jnp.mean(y, axis=-1, keepdims=True)
    v = jnp.mean((y - mu) ** 2, axis=-1, keepdims=True)
    return (y - mu) * lax.rsqrt(v + 1e-5) * g + b


def _dot(a, b):
    return jnp.dot(a, b, preferred_element_type=jnp.float32)


_BE = 2560
_BN = 2000


def _row_spec(blk, width):
    return pl.BlockSpec((blk, width), lambda i: (i, 0))


def _rep_spec(shape):
    nd = len(shape)
    return pl.BlockSpec(shape, lambda i: (0,) * nd)


def _enc_e_body(ea, W1, b1, W2, b2, g, bt, out):
    h = jnp.maximum(_dot(ea[...], W1[...]) + b1[...], 0.0)
    y = _dot(h, W2[...]) + b2[...]
    out[...] = _ln(y, g[...], bt[...])


def _enc_n_body(na, W1, b1, W2, b2, g, bt, W1s, W1d, x_out, xs_out, xd_out):
    h = jnp.maximum(_dot(na[...], W1[...]) + b1[...], 0.0)
    y = _dot(h, W2[...]) + b2[...]
    x = _ln(y, g[...], bt[...])
    x_out[...] = x
    xs_out[...] = _dot(x, W1s[...])
    xd_out[...] = _dot(x, W1d[...])


def _edge_mlp_body(e, gs, gd, W1e, b1, W2, b2, g, bt, out):
    ev = e[...]
    h = jnp.maximum(_dot(ev, W1e[...]) + gs[...] + gd[...] + b1[...], 0.0)
    y = _dot(h, W2[...]) + b2[...]
    out[...] = ev + _ln(y, g[...], bt[...])


def _node_mlp_next_body(x, a0, a1, W1x, W1a, b1, W2, b2, g, bt, W1s, W1d,
                        x_out, xs_out, xd_out):
    xv = x[...]
    agg = a0[...] + a1[...]
    h = jnp.maximum(_dot(xv, W1x[...]) + _dot(agg, W1a[...]) + b1[...], 0.0)
    y = _dot(h, W2[...]) + b2[...]
    xn = xv + _ln(y, g[...], bt[...])
    x_out[...] = xn
    xs_out[...] = _dot(xn, W1s[...])
    xd_out[...] = _dot(xn, W1d[...])


def _node_mlp_last_body(x, a0, a1, W1x, W1a, b1, W2, b2, g, bt, x_out):
    xv = x[...]
    agg = a0[...] + a1[...]
    h = jnp.maximum(_dot(xv, W1x[...]) + _dot(agg, W1a[...]) + b1[...], 0.0)
    y = _dot(h, W2[...]) + b2[...]
    x_out[...] = xv + _ln(y, g[...], bt[...])


def _dec_body(x, W1, b1, W2, b2, out):
    h = jnp.maximum(_dot(x[...], W1[...]) + b1[...], 0.0)
    out[...] = _dot(h, W2[...]) + b2[...]


def _f32(shape):
    return jax.ShapeDtypeStruct(shape, jnp.float32)


def _row1(v):
    return v.reshape(1, -1)


def kernel(node_attr, edge_attr, edge_index, params):
    p = params
    src2 = edge_index[0].reshape(_NW, _NCH, _CH)
    dst2 = edge_index[1].reshape(_NW, _NCH, _CH)
    zeros = jnp.zeros((_NP, _H), jnp.float32)

    wspec = _rep_spec((_H, _H))
    vspec = _rep_spec((1, _H))

    enc_n = pl.pallas_call(
        _enc_n_body,
        grid=(_N // _BN,),
        in_specs=[_row_spec(_BN, _H)] + [wspec, vspec, wspec, vspec, vspec,
                                         vspec, wspec, wspec],
        out_specs=[_row_spec(_BN, _H)] * 3,
        out_shape=[_f32((_N, _H))] * 3,
    )
    x, xs, xd = enc_n(
        node_attr,
        p['enc_n_W1'], _row1(p['enc_n_b1']), p['enc_n_W2'], _row1(p['enc_n_b2']),
        _row1(p['enc_n_g']), _row1(p['enc_n_beta']),
        p['pe_W1'][0, _H:2 * _H], p['pe_W1'][0, 2 * _H:],
    )

    enc_e = pl.pallas_call(
        _enc_e_body,
        grid=(_E // _BE,),
        in_specs=[_row_spec(_BE, 4), _rep_spec((4, _H)), vspec, wspec, vspec,
                  vspec, vspec],
        out_specs=_row_spec(_BE, _H),
        out_shape=_f32((_E, _H)),
    )
    e = enc_e(
        edge_attr,
        p['enc_e_W1'], _row1(p['enc_e_b1']), p['enc_e_W2'], _row1(p['enc_e_b2']),
        _row1(p['enc_e_g']), _row1(p['enc_e_beta']),
    )

    edge_mlp = pl.pallas_call(
        _edge_mlp_body,
        grid=(_E // _BE,),
        in_specs=[_row_spec(_BE, _H)] * 3 + [wspec, vspec, wspec, vspec,
                                             vspec, vspec],
        out_specs=_row_spec(_BE, _H),
        out_shape=_f32((_E, _H)),
    )
    node_mlp_next = pl.pallas_call(
        _node_mlp_next_body,
        grid=(_N // _BN,),
        in_specs=[_row_spec(_BN, _H)] * 3 + [wspec, wspec, vspec, wspec,
                                             vspec, vspec, vspec, wspec, wspec],
        out_specs=[_row_spec(_BN, _H)] * 3,
        out_shape=[_f32((_N, _H))] * 3,
    )
    node_mlp_last = pl.pallas_call(
        _node_mlp_last_body,
        grid=(_N // _BN,),
        in_specs=[_row_spec(_BN, _H)] * 3 + [wspec, wspec, vspec, wspec,
                                             vspec, vspec, vspec],
        out_specs=_row_spec(_BN, _H),
        out_shape=_f32((_N, _H)),
    )

    for i in range(_L):
        gs, gd = _sc_gather(xs, xd, src2, dst2)
        e = edge_mlp(
            e, gs, gd,
            p['pe_W1'][i, :_H], _row1(p['pe_b1'][i]),
            p['pe_W2'][i], _row1(p['pe_b2'][i]),
            _row1(p['pe_g'][i]), _row1(p['pe_beta'][i]),
        )
        agg2 = _sc_scatter(e, dst2, zeros)
        if i < _L - 1:
            x, xs, xd = node_mlp_next(
                x, agg2[0, :_N], agg2[1, :_N],
                p['pn_W1'][i, :_H], p['pn_W1'][i, _H:], _row1(p['pn_b1'][i]),
                p['pn_W2'][i], _row1(p['pn_b2'][i]),
                _row1(p['pn_g'][i]), _row1(p['pn_beta'][i]),
                p['pe_W1'][i + 1, _H:2 * _H], p['pe_W1'][i + 1, 2 * _H:],
            )
        else:
            x = node_mlp_last(
                x, agg2[0, :_N], agg2[1, :_N],
                p['pn_W1'][i, :_H], p['pn_W1'][i, _H:], _row1(p['pn_b1'][i]),
                p['pn_W2'][i], _row1(p['pn_b2'][i]),
                _row1(p['pn_g'][i]), _row1(p['pn_beta'][i]),
            )

    dec = pl.pallas_call(
        _dec_body,
        grid=(_N // _BN,),
        in_specs=[_row_spec(_BN, _H), wspec, vspec, _rep_spec((_H, 3)),
                  _rep_spec((1, 3))],
        out_specs=_row_spec(_BN, 3),
        out_shape=_f32((_N, 3)),
    )
    return dec(x, p['dec_W1'], _row1(p['dec_b1']), p['dec_W2'],
               _row1(p['dec_b2']))


# speedup vs baseline: 4.1434x; 1.0343x over previous
import functools

import jax
import jax.numpy as jnp
from jax import lax
from jax.experimental import pallas as pl
from jax.experimental.pallas import tpu as pltpu
from jax.experimental.pallas import tpu_sc as plsc

_N = 10000
_E = 320000
_H = 128
_L = 15

_NC = 2
_NS = 16
_NW = _NC * _NS
_EPW = _E // _NW
_CH = 80
_NCH = _EPW // _CH
_NP = 10240
_RPT = _NP // _NS

_mesh = plsc.VectorSubcoreMesh(core_axis_name="c", subcore_axis_name="s")


def _wid():
    return lax.axis_index("s") * _NC + lax.axis_index("c")


@functools.partial(
    pl.kernel,
    out_type=jax.ShapeDtypeStruct((_E, _H), jnp.float32),
    mesh=_mesh,
    scratch_types=[
        pltpu.VMEM((_NCH, _CH), jnp.int32),
        pltpu.VMEM((_NCH, _CH), jnp.int32),
        pltpu.VMEM((_CH, _H), jnp.float32),
        pltpu.VMEM((_CH, _H), jnp.float32),
        pltpu.VMEM((_CH, _H), jnp.float32),
        pltpu.VMEM((_CH, _H), jnp.float32),
        pltpu.SemaphoreType.DMA,
        pltpu.SemaphoreType.DMA,
    ],
)
def _sc_gather(xs_hbm, xd_hbm, src_hbm, dst_hbm, g_hbm,
               sidx, didx, rs0, rd0, rs1, rd1, gsem, wsem):
    w = _wid()
    pltpu.sync_copy(src_hbm.at[w], sidx)
    pltpu.sync_copy(dst_hbm.at[w], didx)
    base = w * _EPW

    def chunk_gather(j, rs, rd):
        cs = pltpu.async_copy(xs_hbm.at[sidx.at[j]], rs, gsem)
        cd = pltpu.async_copy(xd_hbm.at[didx.at[j]], rd, gsem)
        return cs, cd

    def chunk_add(rs, rd):
        def body(r):
            for k in range(_H // 16):
                sl = pl.ds(k * 16, 16)
                rs[r, sl] = rs[r, sl] + rd[r, sl]
        plsc.parallel_loop(0, _CH, unroll=2)(body)

    def chunk_write(j, rs):
        return pltpu.async_copy(rs, g_hbm.at[pl.ds(base + j * _CH, _CH)], wsem)

    def body(i, carry):
        j0 = i * 2
        j1 = j0 + 1
        cs0, cd0 = chunk_gather(j0, rs0, rd0)
        cs1, cd1 = chunk_gather(j1, rs1, rd1)
        cs0.wait()
        cd0.wait()
        chunk_add(rs0, rd0)
        w0 = chunk_write(j0, rs0)
        cs1.wait()
        cd1.wait()
        chunk_add(rs1, rd1)
        w1 = chunk_write(j1, rs1)
        w0.wait()
        w1.wait()
        return carry

    lax.fori_loop(0, _NCH // 2, body, 0)
    jl = _NCH - 1
    cs, cd = chunk_gather(jl, rs0, rd0)
    cs.wait()
    cd.wait()
    chunk_add(rs0, rd0)
    chunk_write(jl, rs0).wait()


@functools.partial(
    pl.kernel,
    out_type=jax.ShapeDtypeStruct((_NC, _NP, _H), jnp.float32),
    mesh=_mesh,
    scratch_types=[
        pltpu.VMEM((_NCH, _CH), jnp.int32),
        pltpu.VMEM((_CH, _H), jnp.float32),
        pltpu.VMEM((_CH, _H), jnp.float32),
        pltpu.VMEM_SHARED((_NP, _H), jnp.float32),
        pltpu.SemaphoreType.DMA,
        pltpu.SemaphoreType.DMA,
    ],
)
def _sc_scatter(e_hbm, dst_hbm, zeros_hbm, out_hbm, didx, rows0, rows1, acc,
                rsem, asem):
    c = lax.axis_index("c")
    s = lax.axis_index("s")
    w = s * _NC + c
    pltpu.sync_copy(zeros_hbm.at[pl.ds(s * _RPT, _RPT)],
                    acc.at[pl.ds(s * _RPT, _RPT)])
    pltpu.sync_copy(dst_hbm.at[w], didx)
    plsc.subcore_barrier()
    base = w * _EPW

    def body(i, carry):
        j0 = i * 2
        j1 = j0 + 1
        r0 = pltpu.async_copy(e_hbm.at[pl.ds(base + j0 * _CH, _CH)], rows0, rsem)
        r1 = pltpu.async_copy(e_hbm.at[pl.ds(base + j1 * _CH, _CH)], rows1, rsem)
        r0.wait()
        a0 = pltpu.async_copy(rows0, acc.at[didx.at[j0]], asem, add=True)
        r1.wait()
        a1 = pltpu.async_copy(rows1, acc.at[didx.at[j1]], asem, add=True)
        a0.wait()
        a1.wait()
        return carry

    lax.fori_loop(0, _NCH // 2, body, 0)
    jl = _NCH - 1
    rl = pltpu.async_copy(e_hbm.at[pl.ds(base + jl * _CH, _CH)], rows0, rsem)
    rl.wait()
    al = pltpu.async_copy(rows0, acc.at[didx.at[jl]], asem, add=True)
    al.wait()
    plsc.subcore_barrier()
    pltpu.sync_copy(acc.at[pl.ds(s * _RPT, _RPT)],
                    out_hbm.at[c, pl.ds(s * _RPT, _RPT)])


def _ln(y, g, b):
    mu = jnp.mean(y, axis=-1, keepdims=True)
    v = jnp.mean((y - mu) ** 2, axis=-1, keepdims=True)
    return (y - mu) / jnp.sqrt(v + 1e-5) * g + b


def _dot(a, b):
    return jnp.dot(a, b, preferred_element_type=jnp.float32)


_BE = 2560
_BN = 2000


def _row_spec(blk, width):
    return pl.BlockSpec((blk, width), lambda i: (i, 0))


def _rep_spec(shape):
    nd = len(shape)
    return pl.BlockSpec(shape, lambda i: (0,) * nd)


def _enc_e_body(ea, W1, b1, W2, b2, g, bt, out):
    h = jnp.maximum(_dot(ea[...], W1[...]) + b1[...], 0.0)
    y = _dot(h, W2[...]) + b2[...]
    out[...] = _ln(y, g[...], bt[...])


def _enc_n_body(na, W1, b1, W2, b2, g, bt, W1s, W1d, x_out, xs_out, xd_out):
    h = jnp.maximum(_dot(na[...], W1[...]) + b1[...], 0.0)
    y = _dot(h, W2[...]) + b2[...]
    x = _ln(y, g[...], bt[...])
    x_out[...] = x
    xs_out[...] = _dot(x, W1s[...])
    xd_out[...] = _dot(x, W1d[...])


def _edge_mlp_body(e, gv, W1e, b1, W2, b2, g, bt, out):
    ev = e[...]
    h = jnp.maximum(_dot(ev, W1e[...]) + gv[...] + b1[...], 0.0)
    y = _dot(h, W2[...]) + b2[...]
    out[...] = ev + _ln(y, g[...], bt[...])


def _node_mlp_next_body(x, a0, a1, W1x, W1a, b1, W2, b2, g, bt, W1s, W1d,
                        x_out, xs_out, xd_out):
    xv = x[...]
    agg = a0[...] + a1[...]
    h = jnp.maximum(_dot(xv, W1x[...]) + _dot(agg, W1a[...]) + b1[...], 0.0)
    y = _dot(h, W2[...]) + b2[...]
    xn = xv + _ln(y, g[...], bt[...])
    x_out[...] = xn
    xs_out[...] = _dot(xn, W1s[...])
    xd_out[...] = _dot(xn, W1d[...])


def _node_mlp_last_body(x, a0, a1, W1x, W1a, b1, W2, b2, g, bt, x_out):
    xv = x[...]
    agg = a0[...] + a1[...]
    h = jnp.maximum(_dot(xv, W1x[...]) + _dot(agg, W1a[...]) + b1[...], 0.0)
    y = _dot(h, W2[...]) + b2[...]
    x_out[...] = xv + _ln(y, g[...], bt[...])


def _dec_body(x, W1, b1, W2, b2, out):
    h = jnp.maximum(_dot(x[...], W1[...]) + b1[...], 0.0)
    out[...] = _dot(h, W2[...]) + b2[...]


def _f32(shape):
    return jax.ShapeDtypeStruct(shape, jnp.float32)


def _row1(v):
    return v.reshape(1, -1)


def kernel(node_attr, edge_attr, edge_index, params):
    p = params
    src2 = edge_index[0].reshape(_NW, _NCH, _CH)
    dst2 = edge_index[1].reshape(_NW, _NCH, _CH)
    zeros = jnp.zeros((_NP, _H), jnp.float32)

    wspec = _rep_spec((_H, _H))
    vspec = _rep_spec((1, _H))

    enc_n = pl.pallas_call(
        _enc_n_body,
        grid=(_N // _BN,),
        in_specs=[_row_spec(_BN, _H)] + [wspec, vspec, wspec, vspec, vspec,
                                         vspec, wspec, wspec],
        out_specs=[_row_spec(_BN, _H)] * 3,
        out_shape=[_f32((_N, _H))] * 3,
    )
    x, xs, xd = enc_n(
        node_attr,
        p['enc_n_W1'], _row1(p['enc_n_b1']), p['enc_n_W2'], _row1(p['enc_n_b2']),
        _row1(p['enc_n_g']), _row1(p['enc_n_beta']),
        p['pe_W1'][0, _H:2 * _H], p['pe_W1'][0, 2 * _H:],
    )

    enc_e = pl.pallas_call(
        _enc_e_body,
        grid=(_E // _BE,),
        in_specs=[_row_spec(_BE, 4), _rep_spec((4, _H)), vspec, wspec, vspec,
                  vspec, vspec],
        out_specs=_row_spec(_BE, _H),
        out_shape=_f32((_E, _H)),
    )
    e = enc_e(
        edge_attr,
        p['enc_e_W1'], _row1(p['enc_e_b1']), p['enc_e_W2'], _row1(p['enc_e_b2']),
        _row1(p['enc_e_g']), _row1(p['enc_e_beta']),
    )

    edge_mlp = pl.pallas_call(
        _edge_mlp_body,
        grid=(_E // _BE,),
        in_specs=[_row_spec(_BE, _H)] * 2 + [wspec, vspec, wspec, vspec,
                                             vspec, vspec],
        out_specs=_row_spec(_BE, _H),
        out_shape=_f32((_E, _H)),
    )
    node_mlp_next = pl.pallas_call(
        _node_mlp_next_body,
        grid=(_N // _BN,),
        in_specs=[_row_spec(_BN, _H)] * 3 + [wspec, wspec, vspec, wspec,
                                             vspec, vspec, vspec, wspec, wspec],
        out_specs=[_row_spec(_BN, _H)] * 3,
        out_shape=[_f32((_N, _H))] * 3,
    )
    node_mlp_last = pl.pallas_call(
        _node_mlp_last_body,
        grid=(_N // _BN,),
        in_specs=[_row_spec(_BN, _H)] * 3 + [wspec, wspec, vspec, wspec,
                                             vspec, vspec, vspec],
        out_specs=_row_spec(_BN, _H),
        out_shape=_f32((_N, _H)),
    )

    for i in range(_L):
        g = _sc_gather(xs, xd, src2, dst2)
        e = edge_mlp(
            e, g,
            p['pe_W1'][i, :_H], _row1(p['pe_b1'][i]),
            p['pe_W2'][i], _row1(p['pe_b2'][i]),
            _row1(p['pe_g'][i]), _row1(p['pe_beta'][i]),
        )
        agg2 = _sc_scatter(e, dst2, zeros)
        if i < _L - 1:
            x, xs, xd = node_mlp_next(
                x, agg2[0, :_N], agg2[1, :_N],
                p['pn_W1'][i, :_H], p['pn_W1'][i, _H:], _row1(p['pn_b1'][i]),
                p['pn_W2'][i], _row1(p['pn_b2'][i]),
                _row1(p['pn_g'][i]), _row1(p['pn_beta'][i]),
                p['pe_W1'][i + 1, _H:2 * _H], p['pe_W1'][i + 1, 2 * _H:],
            )
        else:
            x = node_mlp_last(
                x, agg2[0, :_N], agg2[1, :_N],
                p['pn_W1'][i, :_H], p['pn_W1'][i, _H:], _row1(p['pn_b1'][i]),
                p['pn_W2'][i], _row1(p['pn_b2'][i]),
                _row1(p['pn_g'][i]), _row1(p['pn_beta'][i]),
            )

    dec = pl.pallas_call(
        _dec_body,
        grid=(_N // _BN,),
        in_specs=[_row_spec(_BN, _H), wspec, vspec, _rep_spec((_H, 3)),
                  _rep_spec((1, 3))],
        out_specs=_row_spec(_BN, 3),
        out_shape=_f32((_N, 3)),
    )
    return dec(x, p['dec_W1'], _row1(p['dec_b1']), p['dec_W2'],
               _row1(p['dec_b2']))
